# baseline (speedup 1.0000x reference)
import numpy as np

import jax
import jax.numpy as jnp
from jax import lax
from jax.experimental import pallas as pl
from jax.experimental.pallas import tpu as pltpu
from jax.experimental.pallas import tpu_sc as plsc

B = 64
T = 577
D = 768
N = T - 1
NUM_REMAIN = N // 4
OUT_T = NUM_REMAIN + 1
TOTAL_ROWS = B * OUT_T

NC, NS = 2, 16
NW = NC * NS
CHUNK = 32
NCHUNK = 9
NBUF = 3
TAIL = 8
NTAILW = (TOTAL_ROWS - NW * CHUNK * NCHUNK) // TAIL


def _rotl32(x, r):
    return (x << np.uint32(r)) | (x >> np.uint32(32 - r))


def _threefry2x32(k0, k1, x0, x1):
    rotations = ((13, 15, 26, 6), (17, 29, 16, 24))
    ks = (np.uint32(k0), np.uint32(k1),
          np.uint32(k0) ^ np.uint32(k1) ^ np.uint32(0x1BD11BDA))
    x0 = x0 + ks[0]
    x1 = x1 + ks[1]
    with np.errstate(over="ignore"):
        for i in range(5):
            for r in rotations[i % 2]:
                x0 = x0 + x1
                x1 = _rotl32(x1, r)
                x1 = x1 ^ x0
            x0 = x0 + ks[(i + 1) % 3]
            x1 = x1 + ks[(i + 2) % 3] + np.uint32(i + 1)
    return x0, x1


def _fixed_uniform_noise(seed, shape):
    size = int(np.prod(shape))
    o0, o1 = _threefry2x32(0, seed, np.zeros(size, np.uint32),
                           np.arange(size, dtype=np.uint32))
    bits = o0 ^ o1
    floats = (bits >> np.uint32(9)) | np.uint32(0x3F800000)
    return (floats.view(np.float32) - np.float32(1.0)).reshape(shape)


def _index_constants():
    noise = _fixed_uniform_noise(42, (B, N))
    shuffle = np.argsort(noise, axis=-1, kind="stable").astype(np.int32)
    revert = np.argsort(shuffle, axis=-1, kind="stable").astype(np.int32)
    remain = shuffle[:, :NUM_REMAIN]
    masked = shuffle[:, NUM_REMAIN:]

    bb = np.arange(B)[None, :]
    src_t = np.zeros((OUT_T, B), np.int32)
    src_t[1:] = 1 + remain.T
    gidx = (src_t * B + bb).reshape(TOTAL_ROWS).astype(np.int32)

    idx = np.zeros((NW, NCHUNK + 1, CHUNK), np.int32)
    for w in range(NW):
        b0 = CHUNK * NCHUNK * w + TAIL * min(w, NTAILW)
        idx[w, :NCHUNK] = gidx[b0:b0 + NCHUNK * CHUNK].reshape(NCHUNK, CHUNK)
        if w < NTAILW:
            idx[w, NCHUNK, :TAIL] = gidx[b0 + NCHUNK * CHUNK:
                                         b0 + NCHUNK * CHUNK + TAIL]
    return remain, masked, revert, idx


_REMAIN, _MASKED, _REVERT, _IDX = _index_constants()


def _gather_kernel(table_hbm, idx_hbm, out_hbm, idx_v,
                   buf0, buf1, buf2, tbuf,
                   gsem0, gsem1, gsem2, semt, wsem0, wsem1, wsem2):
    wid = lax.axis_index("s") * NC + lax.axis_index("c")
    base = CHUNK * NCHUNK * wid + TAIL * jnp.minimum(wid, NTAILW)
    pltpu.sync_copy(idx_hbm.at[wid], idx_v)

    bufs = (buf0, buf1, buf2)
    gsems = (gsem0, gsem1, gsem2)
    wsems = (wsem0, wsem1, wsem2)
    has_tail = wid < NTAILW

    def gather(c, s):
        return pltpu.make_async_copy(table_hbm.at[idx_v.at[c]], bufs[s],
                                     gsems[s])

    def write(c, s):
        return pltpu.make_async_copy(
            bufs[s], out_hbm.at[pl.ds(base + c * CHUNK, CHUNK)], wsems[s])

    def tail_gather():
        return pltpu.make_async_copy(
            table_hbm.at[idx_v.at[NCHUNK, pl.ds(0, TAIL)]], tbuf, semt)

    for k in range(NBUF):
        gather(k, k).start()
    for c in range(NCHUNK):
        s = c % NBUF
        gather(c, s).wait()
        write(c, s).start()
        n = c + NBUF
        if n < NCHUNK:
            write(n - NBUF, s).wait()
            gather(n, s).start()
        elif n == NCHUNK:
            @pl.when(has_tail)
            def _():
                tail_gather().start()

    @pl.when(has_tail)
    def _():
        tail_gather().wait()
        pltpu.sync_copy(tbuf, out_hbm.at[pl.ds(base + NCHUNK * CHUNK, TAIL)])

    for c in range(max(0, NCHUNK - NBUF), NCHUNK):
        write(c, c % NBUF).wait()


@jax.jit
def _run(data):
    table = data.transpose(1, 0, 2).reshape(T * B, D)

    small = (jnp.asarray(_REMAIN), jnp.asarray(_MASKED), jnp.asarray(_REVERT),
             jnp.ones((B, OUT_T), dtype=jnp.float32),
             jnp.ones((B, T), dtype=jnp.float32))
    table, small = lax.optimization_barrier((table, small))
    remain_idx, masked_idx, revert_idx, remain_padding_mask, \
        revert_padding_mask = small

    mesh = plsc.VectorSubcoreMesh(core_axis_name="c", subcore_axis_name="s")
    flat_out = pl.kernel(
        _gather_kernel,
        mesh=mesh,
        out_type=jax.ShapeDtypeStruct((TOTAL_ROWS, D), jnp.float32),
        scratch_types=[
            pltpu.VMEM((NCHUNK + 1, CHUNK), jnp.int32),
            pltpu.VMEM((CHUNK, D), jnp.float32),
            pltpu.VMEM((CHUNK, D), jnp.float32),
            pltpu.VMEM((CHUNK, D), jnp.float32),
            pltpu.VMEM((TAIL, D), jnp.float32),
            pltpu.SemaphoreType.DMA,
            pltpu.SemaphoreType.DMA,
            pltpu.SemaphoreType.DMA,
            pltpu.SemaphoreType.DMA,
            pltpu.SemaphoreType.DMA,
            pltpu.SemaphoreType.DMA,
            pltpu.SemaphoreType.DMA,
        ],
    )(table, jnp.asarray(_IDX))

    img_remain = flat_out.reshape(OUT_T, B, D).transpose(1, 0, 2)
    return (img_remain, remain_idx, masked_idx, revert_idx,
            remain_padding_mask, revert_padding_mask)


def kernel(data):
    return _run(data)


# speedup vs baseline: 1.0105x; 1.0105x over previous
import numpy as np

import jax
import jax.numpy as jnp
from jax import lax
from jax.experimental import pallas as pl
from jax.experimental.pallas import tpu as pltpu
from jax.experimental.pallas import tpu_sc as plsc

B = 64
T = 577
D = 768
N = T - 1
NUM_REMAIN = N // 4
OUT_T = NUM_REMAIN + 1
TOTAL_ROWS = B * OUT_T

NC, NS = 2, 16
NW = NC * NS
CHUNK = 48
NCHUNK = 6
NBUF = 3
TAIL = 8
NTAILW = (TOTAL_ROWS - NW * CHUNK * NCHUNK) // TAIL


def _rotl32(x, r):
    return (x << np.uint32(r)) | (x >> np.uint32(32 - r))


def _threefry2x32(k0, k1, x0, x1):
    rotations = ((13, 15, 26, 6), (17, 29, 16, 24))
    ks = (np.uint32(k0), np.uint32(k1),
          np.uint32(k0) ^ np.uint32(k1) ^ np.uint32(0x1BD11BDA))
    x0 = x0 + ks[0]
    x1 = x1 + ks[1]
    with np.errstate(over="ignore"):
        for i in range(5):
            for r in rotations[i % 2]:
                x0 = x0 + x1
                x1 = _rotl32(x1, r)
                x1 = x1 ^ x0
            x0 = x0 + ks[(i + 1) % 3]
            x1 = x1 + ks[(i + 2) % 3] + np.uint32(i + 1)
    return x0, x1


def _fixed_uniform_noise(seed, shape):
    size = int(np.prod(shape))
    o0, o1 = _threefry2x32(0, seed, np.zeros(size, np.uint32),
                           np.arange(size, dtype=np.uint32))
    bits = o0 ^ o1
    floats = (bits >> np.uint32(9)) | np.uint32(0x3F800000)
    return (floats.view(np.float32) - np.float32(1.0)).reshape(shape)


def _index_constants():
    noise = _fixed_uniform_noise(42, (B, N))
    shuffle = np.argsort(noise, axis=-1, kind="stable").astype(np.int32)
    revert = np.argsort(shuffle, axis=-1, kind="stable").astype(np.int32)
    remain = shuffle[:, :NUM_REMAIN]
    masked = shuffle[:, NUM_REMAIN:]

    bb = np.arange(B)[None, :]
    src_t = np.zeros((OUT_T, B), np.int32)
    src_t[1:] = 1 + remain.T
    gidx = (src_t * B + bb).reshape(TOTAL_ROWS).astype(np.int32)

    idx = np.zeros((NW, NCHUNK + 1, CHUNK), np.int32)
    for w in range(NW):
        b0 = CHUNK * NCHUNK * w + TAIL * min(w, NTAILW)
        idx[w, :NCHUNK] = gidx[b0:b0 + NCHUNK * CHUNK].reshape(NCHUNK, CHUNK)
        if w < NTAILW:
            idx[w, NCHUNK, :TAIL] = gidx[b0 + NCHUNK * CHUNK:
                                         b0 + NCHUNK * CHUNK + TAIL]
    return remain, masked, revert, idx


_REMAIN, _MASKED, _REVERT, _IDX = _index_constants()


def _gather_kernel(table_hbm, idx_hbm, out_hbm, idx_v,
                   buf0, buf1, buf2, tbuf,
                   gsem0, gsem1, gsem2, semt, wsem0, wsem1, wsem2):
    wid = lax.axis_index("s") * NC + lax.axis_index("c")
    base = CHUNK * NCHUNK * wid + TAIL * jnp.minimum(wid, NTAILW)
    pltpu.sync_copy(idx_hbm.at[wid], idx_v)

    bufs = (buf0, buf1, buf2)
    gsems = (gsem0, gsem1, gsem2)
    wsems = (wsem0, wsem1, wsem2)
    has_tail = wid < NTAILW

    def gather(c, s):
        return pltpu.make_async_copy(table_hbm.at[idx_v.at[c]], bufs[s],
                                     gsems[s])

    def write(c, s):
        return pltpu.make_async_copy(
            bufs[s], out_hbm.at[pl.ds(base + c * CHUNK, CHUNK)], wsems[s])

    def tail_gather():
        return pltpu.make_async_copy(
            table_hbm.at[idx_v.at[NCHUNK, pl.ds(0, TAIL)]], tbuf, semt)

    for k in range(NBUF):
        gather(k, k).start()
    for c in range(NCHUNK):
        s = c % NBUF
        gather(c, s).wait()
        write(c, s).start()
        n = c + NBUF
        if n < NCHUNK:
            write(n - NBUF, s).wait()
            gather(n, s).start()
        elif n == NCHUNK:
            @pl.when(has_tail)
            def _():
                tail_gather().start()

    @pl.when(has_tail)
    def _():
        tail_gather().wait()
        pltpu.sync_copy(tbuf, out_hbm.at[pl.ds(base + NCHUNK * CHUNK, TAIL)])

    for c in range(max(0, NCHUNK - NBUF), NCHUNK):
        write(c, c % NBUF).wait()


@jax.jit
def _run(data):
    table = data.transpose(1, 0, 2).reshape(T * B, D)

    small = (jnp.asarray(_REMAIN), jnp.asarray(_MASKED), jnp.asarray(_REVERT),
             jnp.ones((B, OUT_T), dtype=jnp.float32),
             jnp.ones((B, T), dtype=jnp.float32))
    table, small = lax.optimization_barrier((table, small))
    remain_idx, masked_idx, revert_idx, remain_padding_mask, \
        revert_padding_mask = small

    mesh = plsc.VectorSubcoreMesh(core_axis_name="c", subcore_axis_name="s")
    flat_out = pl.kernel(
        _gather_kernel,
        mesh=mesh,
        out_type=jax.ShapeDtypeStruct((TOTAL_ROWS, D), jnp.float32),
        scratch_types=[
            pltpu.VMEM((NCHUNK + 1, CHUNK), jnp.int32),
            pltpu.VMEM((CHUNK, D), jnp.float32),
            pltpu.VMEM((CHUNK, D), jnp.float32),
            pltpu.VMEM((CHUNK, D), jnp.float32),
            pltpu.VMEM((TAIL, D), jnp.float32),
            pltpu.SemaphoreType.DMA,
            pltpu.SemaphoreType.DMA,
            pltpu.SemaphoreType.DMA,
            pltpu.SemaphoreType.DMA,
            pltpu.SemaphoreType.DMA,
            pltpu.SemaphoreType.DMA,
            pltpu.SemaphoreType.DMA,
        ],
    )(table, jnp.asarray(_IDX))

    img_remain = flat_out.reshape(OUT_T, B, D).transpose(1, 0, 2)
    return (img_remain, remain_idx, masked_idx, revert_idx,
            remain_padding_mask, revert_padding_mask)


def kernel(data):
    return _run(data)
